# baseline (speedup 1.0000x reference)
import jax
import jax.numpy as jnp
from jax.experimental import pallas as pl
from jax.experimental.pallas import tpu as pltpu

_ROWS = 16384
_COLS = 2048
_BLOCK_ROWS = 1024
_NB = _ROWS // _BLOCK_ROWS
_K = 2


def _quant_body(nl_ref, x_ref, o_ref, res_ref, accmin_ref, accmax_ref,
                mm_ref):
    p = pl.program_id(0)
    i = pl.program_id(1)

    @pl.when(p == 0)
    def _reduce_phase():
        @pl.when(i == 0)
        def _init():
            accmin_ref[...] = jnp.full((8, _COLS), 3.4e38, jnp.float32)
            accmax_ref[...] = jnp.full((8, _COLS), -3.4e38, jnp.float32)

        x = x_ref[...]
        mn = accmin_ref[...]
        mx = accmax_ref[...]
        for u in range(_BLOCK_ROWS // 8):
            s = x[u * 8:(u + 1) * 8, :]
            mn = jnp.minimum(mn, s)
            mx = jnp.maximum(mx, s)
        accmin_ref[...] = mn
        accmax_ref[...] = mx

        @pl.when(i < _K)
        def _stash():
            res_ref[pl.ds(i * _BLOCK_ROWS, _BLOCK_ROWS), :] = x

    @pl.when(p == 1)
    def _quantize_phase():
        @pl.when(i == 0)
        def _finalize():
            mm_ref[0] = jnp.min(accmin_ref[...])
            mm_ref[1] = jnp.max(accmax_ref[...])

        nl = nl_ref[0]
        rng = mm_ref[1] - mm_ref[0]
        scale = rng / nl
        inv_scale = nl / rng

        @pl.when(i < _K)
        def _from_vmem():
            r = res_ref[pl.ds(i * _BLOCK_ROWS, _BLOCK_ROWS), :]
            o_ref[...] = jnp.round(r * inv_scale) * scale

        @pl.when(i >= _K)
        def _from_hbm():
            o_ref[...] = jnp.round(x_ref[...] * inv_scale) * scale


def kernel(input, bits):
    nlevels = (jnp.exp2(bits.astype(jnp.float32)) - 1.0
               if hasattr(bits, "astype")
               else jnp.float32(2.0 ** bits - 1.0))
    nlevels = jnp.reshape(nlevels, (1,))
    x2 = input.reshape(_ROWS, _COLS)

    def x_map(p, i):
        return (jnp.where(p == 0, i, jnp.where(i < _K, _NB - 1, i)), 0)

    out = pl.pallas_call(
        _quant_body,
        grid=(2, _NB),
        in_specs=[
            pl.BlockSpec(memory_space=pltpu.SMEM),
            pl.BlockSpec((_BLOCK_ROWS, _COLS), x_map),
        ],
        out_specs=pl.BlockSpec((_BLOCK_ROWS, _COLS), lambda p, i: (p * i, 0)),
        out_shape=jax.ShapeDtypeStruct((_ROWS, _COLS), jnp.float32),
        scratch_shapes=[pltpu.VMEM((_K * _BLOCK_ROWS, _COLS), jnp.float32),
                        pltpu.VMEM((8, _COLS), jnp.float32),
                        pltpu.VMEM((8, _COLS), jnp.float32),
                        pltpu.SMEM((2,), jnp.float32)],
    )(nlevels, x2)
    return out.reshape(input.shape)
